# baseline (speedup 1.0000x reference)
import jax
import jax.numpy as jnp
from jax import lax
from jax.experimental import pallas as pl
from jax.experimental.pallas import tpu as pltpu
from jax.experimental.pallas import tpu_sc as plsc

MARGIN = 1.0
LANES = 16
CH = 208
SUB = 104
NSUB = CH // SUB
NW = 32


def _sc_distance_body(tri, ent, rel, pos_out, neg_out, part_out,
                      tri_v, dist_v, neg_v, pos_v, loss_v,
                      ih0, ir0, it0, ih1, ir1, it1,
                      h0, r0, t0, h1, r1, t1,
                      sem_in, sem_g0, sem_g1):
    dim = ent.shape[1]
    total = tri.shape[0] // 3
    nc = lax.axis_size("c")
    nw = nc * lax.axis_size("s")
    wid = lax.axis_index("s") * nc + lax.axis_index("c")
    rpw = total // nw
    npos = rpw // 65
    nch = rpw // CH
    pairs = nch // 2
    w_base = wid * rpw

    row_ids = lax.iota(jnp.int32, LANES)

    pltpu.async_copy(tri.at[pl.ds(w_base * 3, rpw * 3)], tri_v, sem_in).wait()

    def build_idx(c, ibufs):
        ihB, irB, itB = ibufs
        for g in range(CH // LANES):
            three = (row_ids + (c * CH + g * LANES)) * 3
            sl = pl.ds(g * LANES, LANES)
            ihB[sl] = plsc.load_gather(tri_v, [three])
            irB[sl] = plsc.load_gather(tri_v, [three + 1])
            itB[sl] = plsc.load_gather(tri_v, [three + 2])

    def fire(ibufs, bufs, sem):
        ihB, irB, itB = ibufs
        hB, rB, tB = bufs
        for j in range(NSUB):
            sl = pl.ds(j * SUB, SUB)
            pltpu.async_copy(ent.at[ihB.at[sl]], hB.at[sl], sem)
            pltpu.async_copy(rel.at[irB.at[sl]], rB.at[sl], sem)
            pltpu.async_copy(ent.at[itB.at[sl]], tB.at[sl], sem)

    def drain(ibufs, bufs, sem):
        ihB, irB, itB = ibufs
        hB, rB, tB = bufs
        for j in range(NSUB):
            sl = pl.ds(j * SUB, SUB)
            pltpu.make_async_copy(ent.at[ihB.at[sl]], hB.at[sl], sem).wait()
            pltpu.make_async_copy(rel.at[irB.at[sl]], rB.at[sl], sem).wait()
            pltpu.make_async_copy(ent.at[itB.at[sl]], tB.at[sl], sem).wait()

    def compute(c, bufs):
        hB, rB, tB = bufs
        for g in range(CH // LANES):
            rows16 = row_ids + (g * LANES)

            def dim_body(d4, acc):
                for k in range(4):
                    col = jnp.full((LANES,), d4 * 4 + k, jnp.int32)
                    h = plsc.load_gather(hB, [rows16, col])
                    r = plsc.load_gather(rB, [rows16, col])
                    t = plsc.load_gather(tB, [rows16, col])
                    acc = acc + jnp.abs(h + r - t)
                return acc

            acc = lax.fori_loop(0, dim // 4, dim_body,
                                jnp.zeros((LANES,), jnp.float32))
            dist_v[pl.ds(c * CH + g * LANES, LANES)] = acc

    ibufs0 = (ih0, ir0, it0)
    ibufs1 = (ih1, ir1, it1)
    bufs0 = (h0, r0, t0)
    bufs1 = (h1, r1, t1)

    build_idx(0, ibufs0)
    fire(ibufs0, bufs0, sem_g0)

    def pair_body(p, carry):
        c0 = p * 2
        build_idx(c0 + 1, ibufs1)
        fire(ibufs1, bufs1, sem_g1)
        drain(ibufs0, bufs0, sem_g0)
        compute(c0, bufs0)

        @pl.when(p + 1 < pairs)
        def _():
            build_idx(c0 + 2, ibufs0)
            fire(ibufs0, bufs0, sem_g0)

        drain(ibufs1, bufs1, sem_g1)
        compute(c0 + 1, bufs1)
        return carry

    lax.fori_loop(0, pairs, pair_body, 0)

    for g in range(npos // LANES):
        b16 = row_ids + (g * LANES)
        pos_v[pl.ds(g * LANES, LANES)] = plsc.load_gather(dist_v, [b16 * 65])

    def loss_body(b, acc):
        off = b * 65
        p = jnp.full((LANES,), dist_v[pl.ds(off, LANES)][0])
        for k in range(4):
            v = dist_v[pl.ds(off + 1 + k * LANES, LANES)]
            neg_v[pl.ds(b * 64 + k * LANES, LANES)] = v
            acc = acc + jnp.maximum(p - v + MARGIN, 0.0)
        return acc

    lacc = lax.fori_loop(0, npos, loss_body, jnp.zeros((LANES,), jnp.float32))
    loss_v[...] = lacc

    pltpu.sync_copy(pos_v, pos_out.at[pl.ds(wid * npos, npos)])
    pltpu.sync_copy(neg_v, neg_out.at[pl.ds(wid * npos * 64, npos * 64)])
    pltpu.sync_copy(loss_v, part_out.at[wid])


def _loss_body(p_ref, denom_ref, loss_ref):
    loss_ref[0, 0] = jnp.sum(p_ref[...]) / denom_ref[0]


@jax.jit
def kernel(triple_matrix, entities_emb, relations_emb):
    b, np1, _ = triple_matrix.shape
    total = b * np1
    neg_n = np1 - 1
    tri_flat = triple_matrix.reshape(total * 3)

    mesh = plsc.VectorSubcoreMesh(core_axis_name="c", subcore_axis_name="s")
    dim = entities_emb.shape[1]
    rpw = total // NW
    pos, neg, partials = pl.kernel(
        _sc_distance_body,
        out_type=(
            jax.ShapeDtypeStruct((b,), jnp.float32),
            jax.ShapeDtypeStruct((b * neg_n,), jnp.float32),
            jax.ShapeDtypeStruct((NW, LANES), jnp.float32),
        ),
        mesh=mesh,
        compiler_params=pltpu.CompilerParams(
            needs_layout_passes=False, use_tc_tiling_on_sc=False),
        scratch_types=[
            pltpu.VMEM((rpw * 3,), jnp.int32),
            pltpu.VMEM((rpw,), jnp.float32),
            pltpu.VMEM((rpw // 65 * 64,), jnp.float32),
            pltpu.VMEM((rpw // 65,), jnp.float32),
            pltpu.VMEM((LANES,), jnp.float32),
            pltpu.VMEM((CH,), jnp.int32),
            pltpu.VMEM((CH,), jnp.int32),
            pltpu.VMEM((CH,), jnp.int32),
            pltpu.VMEM((CH,), jnp.int32),
            pltpu.VMEM((CH,), jnp.int32),
            pltpu.VMEM((CH,), jnp.int32),
            pltpu.VMEM((CH, dim), jnp.float32),
            pltpu.VMEM((CH, dim), jnp.float32),
            pltpu.VMEM((CH, dim), jnp.float32),
            pltpu.VMEM((CH, dim), jnp.float32),
            pltpu.VMEM((CH, dim), jnp.float32),
            pltpu.VMEM((CH, dim), jnp.float32),
            pltpu.SemaphoreType.DMA,
            pltpu.SemaphoreType.DMA,
            pltpu.SemaphoreType.DMA,
        ],
    )(tri_flat, entities_emb, relations_emb)

    denom = jnp.full((1,), float(b * neg_n), jnp.float32)
    loss = pl.pallas_call(
        _loss_body,
        out_shape=jax.ShapeDtypeStruct((1, 1), jnp.float32),
        in_specs=[pl.BlockSpec(memory_space=pltpu.VMEM),
                  pl.BlockSpec(memory_space=pltpu.SMEM)],
        out_specs=pl.BlockSpec(memory_space=pltpu.SMEM),
    )(partials, denom)[0, 0]

    return (loss, pos, neg.reshape(b, neg_n))


# speedup vs baseline: 1.5407x; 1.5407x over previous
import jax
import jax.numpy as jnp
from jax import lax
from jax.experimental import pallas as pl
from jax.experimental.pallas import tpu as pltpu
from jax.experimental.pallas import tpu_sc as plsc

MARGIN = 1.0
LANES = 16
BW = 128
NW = 32


def _sc_distance_body(h_t, r_t, t_t, ent, rel, pos_out, negt_out, part_out,
                      ihv, irv, itv, distt_v, loss_v,
                      h0, r0, t0, h1, r1, t1,
                      sem_in, sem_g0, sem_g1):
    np1 = h_t.shape[0]
    nc = lax.axis_size("c")
    wid = lax.axis_index("s") * nc + lax.axis_index("c")
    b0 = wid * BW

    row_ids = lax.iota(jnp.int32, LANES)

    cps = [pltpu.async_copy(src.at[:, pl.ds(b0, BW)], dst, sem_in)
           for src, dst in ((h_t, ihv), (r_t, irv), (t_t, itv))]
    for cp in cps:
        cp.wait()

    def fire(n, bufs, sem):
        hB, rB, tB = bufs
        pltpu.async_copy(ent.at[ihv.at[n]], hB, sem)
        pltpu.async_copy(rel.at[irv.at[n]], rB, sem)
        pltpu.async_copy(ent.at[itv.at[n]], tB, sem)

    def drain(bufs, sem):
        hB, rB, tB = bufs
        pltpu.make_async_copy(ent.at[ihv.at[0]], hB, sem).wait()
        pltpu.make_async_copy(rel.at[irv.at[0]], rB, sem).wait()
        pltpu.make_async_copy(ent.at[itv.at[0]], tB, sem).wait()

    def compute(n, bufs):
        hB, rB, tB = bufs

        def gbody(g, carry):
            vec = jnp.zeros((LANES,), jnp.float32)
            for r in range(LANES):
                l = g * LANES + r
                acc = None
                for k in range(4):
                    sl = pl.ds(k * LANES, LANES)
                    a = jnp.abs(hB[l, sl] + rB[l, sl] - tB[l, sl])
                    acc = a if acc is None else acc + a
                s = jnp.sum(acc)
                vec = jnp.where(row_ids == r, s, vec)
            distt_v[n, pl.ds(g * LANES, LANES)] = vec
            return carry

        lax.fori_loop(0, BW // LANES, gbody, 0)

    bufs0 = (h0, r0, t0)
    bufs1 = (h1, r1, t1)

    fire(0, bufs0, sem_g0)

    def pair_body(p, carry):
        n0 = p * 2
        fire(n0 + 1, bufs1, sem_g1)
        drain(bufs0, sem_g0)
        compute(n0, bufs0)
        fire(n0 + 2, bufs0, sem_g0)
        drain(bufs1, sem_g1)
        compute(n0 + 1, bufs1)
        return carry

    lax.fori_loop(0, (np1 - 1) // 2, pair_body, 0)
    drain(bufs0, sem_g0)
    compute(np1 - 1, bufs0)

    def loss_body(n, acc):
        for g in range(BW // LANES):
            sl = pl.ds(g * LANES, LANES)
            acc = acc + jnp.maximum(distt_v[0, sl] - distt_v[n, sl] + MARGIN,
                                    0.0)
        return acc

    lacc = lax.fori_loop(1, np1, loss_body, jnp.zeros((LANES,), jnp.float32))
    loss_v[...] = lacc

    pltpu.sync_copy(distt_v.at[0], pos_out.at[pl.ds(b0, BW)])
    pltpu.sync_copy(distt_v.at[pl.ds(1, np1 - 1)],
                    negt_out.at[:, pl.ds(b0, BW)])
    pltpu.sync_copy(loss_v, part_out.at[wid])


def _loss_body(p_ref, denom_ref, loss_ref):
    loss_ref[0, 0] = jnp.sum(p_ref[...]) / denom_ref[0]


@jax.jit
def kernel(triple_matrix, entities_emb, relations_emb):
    b, np1, _ = triple_matrix.shape
    neg_n = np1 - 1

    tri_t = jnp.transpose(triple_matrix, (2, 1, 0))
    h_t = tri_t[0]
    r_t = tri_t[1]
    t_t = tri_t[2]

    mesh = plsc.VectorSubcoreMesh(core_axis_name="c", subcore_axis_name="s")
    dim = entities_emb.shape[1]
    pos, negt, partials = pl.kernel(
        _sc_distance_body,
        out_type=(
            jax.ShapeDtypeStruct((b,), jnp.float32),
            jax.ShapeDtypeStruct((neg_n, b), jnp.float32),
            jax.ShapeDtypeStruct((NW, LANES), jnp.float32),
        ),
        mesh=mesh,
        compiler_params=pltpu.CompilerParams(
            needs_layout_passes=False, use_tc_tiling_on_sc=False),
        scratch_types=[
            pltpu.VMEM((np1, BW), jnp.int32),
            pltpu.VMEM((np1, BW), jnp.int32),
            pltpu.VMEM((np1, BW), jnp.int32),
            pltpu.VMEM((np1, BW), jnp.float32),
            pltpu.VMEM((LANES,), jnp.float32),
            pltpu.VMEM((BW, dim), jnp.float32),
            pltpu.VMEM((BW, dim), jnp.float32),
            pltpu.VMEM((BW, dim), jnp.float32),
            pltpu.VMEM((BW, dim), jnp.float32),
            pltpu.VMEM((BW, dim), jnp.float32),
            pltpu.VMEM((BW, dim), jnp.float32),
            pltpu.SemaphoreType.DMA,
            pltpu.SemaphoreType.DMA,
            pltpu.SemaphoreType.DMA,
        ],
    )(h_t, r_t, t_t, entities_emb, relations_emb)

    denom = jnp.full((1,), float(b * neg_n), jnp.float32)
    loss = pl.pallas_call(
        _loss_body,
        out_shape=jax.ShapeDtypeStruct((1, 1), jnp.float32),
        in_specs=[pl.BlockSpec(memory_space=pltpu.VMEM),
                  pl.BlockSpec(memory_space=pltpu.SMEM)],
        out_specs=pl.BlockSpec(memory_space=pltpu.SMEM),
    )(partials, denom)[0, 0]

    return (loss, pos, negt.T)


# speedup vs baseline: 1.5529x; 1.0080x over previous
import jax
import jax.numpy as jnp
from jax import lax
from jax.experimental import pallas as pl
from jax.experimental.pallas import tpu as pltpu
from jax.experimental.pallas import tpu_sc as plsc

MARGIN = 1.0
LANES = 16
BW = 128
CW = 64
NW = 32
PADDIM = 128


def _sc_distance_body(h_t, r_t, t_t, ent, rel, pos_out, negt_out, part_out,
                      ihv, irv, itv, distt_v, loss_v,
                      h0, r0, t0, h1, r1, t1,
                      sem_in, sem_g0, sem_g1):
    np1 = h_t.shape[0]
    dim = 64
    nc = lax.axis_size("c")
    wid = lax.axis_index("s") * nc + lax.axis_index("c")
    b0 = wid * BW

    row_ids = lax.iota(jnp.int32, LANES)

    cps = [pltpu.async_copy(src.at[:, pl.ds(b0, BW)], dst, sem_in)
           for src, dst in ((h_t, ihv), (r_t, irv), (t_t, itv))]
    for cp in cps:
        cp.wait()

    def fire(c, bufs, sem):
        n = c // 2
        j = (c % 2) * CW
        hB, rB, tB = bufs
        pltpu.async_copy(ent.at[ihv.at[n, pl.ds(j, CW)]], hB, sem)
        pltpu.async_copy(rel.at[irv.at[n, pl.ds(j, CW)]], rB, sem)
        pltpu.async_copy(ent.at[itv.at[n, pl.ds(j, CW)]], tB, sem)

    def drain(bufs, sem):
        hB, rB, tB = bufs
        pltpu.make_async_copy(ent.at[ihv.at[0, pl.ds(0, CW)]], hB, sem).wait()
        pltpu.make_async_copy(rel.at[irv.at[0, pl.ds(0, CW)]], rB, sem).wait()
        pltpu.make_async_copy(ent.at[itv.at[0, pl.ds(0, CW)]], tB, sem).wait()

    def compute(c, bufs):
        n = c // 2
        joff = (c % 2) * CW
        hB, rB, tB = bufs

        def gbody(g, carry):
            vec = jnp.zeros((LANES,), jnp.float32)
            for r in range(LANES):
                l = g * LANES + r
                acc = None
                for k in range(dim // LANES):
                    sl = pl.ds(k * LANES, LANES)
                    a = jnp.abs(hB[l, sl] + rB[l, sl] - tB[l, sl])
                    acc = a if acc is None else acc + a
                s = jnp.sum(acc)
                vec = jnp.where(row_ids == r, s, vec)
            distt_v[n, pl.ds(joff + g * LANES, LANES)] = vec
            return carry

        lax.fori_loop(0, CW // LANES, gbody, 0)

    bufs0 = (h0, r0, t0)
    bufs1 = (h1, r1, t1)
    n_chunks = np1 * 2

    fire(0, bufs0, sem_g0)

    def pair_body(p, carry):
        c0 = p * 2
        fire(c0 + 1, bufs1, sem_g1)
        drain(bufs0, sem_g0)
        compute(c0, bufs0)

        @pl.when(p + 1 < n_chunks // 2)
        def _():
            fire(c0 + 2, bufs0, sem_g0)

        drain(bufs1, sem_g1)
        compute(c0 + 1, bufs1)
        return carry

    lax.fori_loop(0, n_chunks // 2, pair_body, 0)

    def loss_body(n, acc):
        for g in range(BW // LANES):
            sl = pl.ds(g * LANES, LANES)
            acc = acc + jnp.maximum(distt_v[0, sl] - distt_v[n, sl] + MARGIN,
                                    0.0)
        return acc

    lacc = lax.fori_loop(1, np1, loss_body, jnp.zeros((LANES,), jnp.float32))
    loss_v[...] = lacc

    pltpu.sync_copy(distt_v.at[0], pos_out.at[pl.ds(b0, BW)])
    pltpu.sync_copy(distt_v.at[pl.ds(1, np1 - 1)],
                    negt_out.at[:, pl.ds(b0, BW)])
    pltpu.sync_copy(loss_v, part_out.at[wid])


def _loss_body(p_ref, denom_ref, loss_ref):
    loss_ref[0, 0] = jnp.sum(p_ref[...]) / denom_ref[0]


@jax.jit
def kernel(triple_matrix, entities_emb, relations_emb):
    b, np1, _ = triple_matrix.shape
    neg_n = np1 - 1
    dim = entities_emb.shape[1]

    tri_t = jnp.transpose(triple_matrix, (2, 1, 0))
    h_t = tri_t[0]
    r_t = tri_t[1]
    t_t = tri_t[2]

    ent128 = jnp.pad(entities_emb, ((0, 0), (0, PADDIM - dim)))
    rel128 = jnp.pad(relations_emb, ((0, 0), (0, PADDIM - dim)))

    mesh = plsc.VectorSubcoreMesh(core_axis_name="c", subcore_axis_name="s")
    pos, negt, partials = pl.kernel(
        _sc_distance_body,
        out_type=(
            jax.ShapeDtypeStruct((b,), jnp.float32),
            jax.ShapeDtypeStruct((neg_n, b), jnp.float32),
            jax.ShapeDtypeStruct((NW, LANES), jnp.float32),
        ),
        mesh=mesh,
        compiler_params=pltpu.CompilerParams(
            needs_layout_passes=False, use_tc_tiling_on_sc=True),
        scratch_types=[
            pltpu.VMEM((np1, BW), jnp.int32),
            pltpu.VMEM((np1, BW), jnp.int32),
            pltpu.VMEM((np1, BW), jnp.int32),
            pltpu.VMEM((np1, BW), jnp.float32),
            pltpu.VMEM((LANES,), jnp.float32),
            pltpu.VMEM((CW, PADDIM), jnp.float32),
            pltpu.VMEM((CW, PADDIM), jnp.float32),
            pltpu.VMEM((CW, PADDIM), jnp.float32),
            pltpu.VMEM((CW, PADDIM), jnp.float32),
            pltpu.VMEM((CW, PADDIM), jnp.float32),
            pltpu.VMEM((CW, PADDIM), jnp.float32),
            pltpu.SemaphoreType.DMA,
            pltpu.SemaphoreType.DMA,
            pltpu.SemaphoreType.DMA,
        ],
    )(h_t, r_t, t_t, ent128, rel128)

    denom = jnp.full((1,), float(b * neg_n), jnp.float32)
    loss = pl.pallas_call(
        _loss_body,
        out_shape=jax.ShapeDtypeStruct((1, 1), jnp.float32),
        in_specs=[pl.BlockSpec(memory_space=pltpu.VMEM),
                  pl.BlockSpec(memory_space=pltpu.SMEM)],
        out_specs=pl.BlockSpec(memory_space=pltpu.SMEM),
    )(partials, denom)[0, 0]

    return (loss, pos, negt.T)
